# Initial kernel scaffold; baseline (speedup 1.0000x reference)
import jax
import jax.numpy as jnp
from jax.experimental import pallas as pl


def kernel(x, pos, edge_index, batch, W1, b1, g1, be1, W2, b2):
    raise NotImplementedError("write your pallas kernel here")


# speedup vs baseline: 5.3741x; 5.3741x over previous
import functools

import jax
import jax.numpy as jnp
from jax import lax
from jax.experimental import pallas as pl
from jax.experimental.pallas import tpu as pltpu
from jax.experimental.pallas import tpu_sc as plsc

_CW = 33
_AW = 24
_NW = 32
_GCH = 512
_SCH = 1024
_TRASH = 48


def _sc_gather(px, py, pz, src1d, dst1d, EP, N):
    nchunk = EP // _GCH
    per_w = nchunk // _NW
    nsub = _GCH // 128
    mesh = plsc.VectorSubcoreMesh(core_axis_name="c", subcore_axis_name="s")

    @functools.partial(
        pl.kernel,
        out_type=(jax.ShapeDtypeStruct((EP,), jnp.float32),
                  jax.ShapeDtypeStruct((EP,), jnp.float32),
                  jax.ShapeDtypeStruct((EP,), jnp.float32)),
        mesh=mesh,
        scratch_types=[
            pltpu.VMEM((_GCH,), jnp.int32),
            pltpu.VMEM((_GCH,), jnp.int32),
            pltpu.VMEM((_GCH,), jnp.float32),
            pltpu.VMEM((_GCH,), jnp.float32),
            pltpu.VMEM((_GCH,), jnp.float32),
            pltpu.VMEM((_GCH,), jnp.float32),
            pltpu.VMEM((_GCH,), jnp.float32),
            pltpu.VMEM((_GCH,), jnp.float32),
            pltpu.SemaphoreType.DMA,
        ],
    )
    def k(px_hbm, py_hbm, pz_hbm, src_hbm, dst_hbm, ox, oy, oz,
          si_v, di_v, sx, sy, sz, dx, dy, dz, sem):
        c = lax.axis_index("c")
        s = lax.axis_index("s")
        wid = s * 2 + c

        def body(t, carry):
            ct = wid * per_w + t
            e0 = ct * _GCH
            pltpu.sync_copy(src_hbm.at[pl.ds(e0, _GCH)], si_v)
            pltpu.sync_copy(dst_hbm.at[pl.ds(e0, _GCH)], di_v)
            cps = []
            for j in range(nsub):
                sl = pl.ds(j * 128, 128)
                cps.append(pltpu.async_copy(px_hbm.at[si_v.at[sl]], sx.at[sl], sem))
                cps.append(pltpu.async_copy(py_hbm.at[si_v.at[sl]], sy.at[sl], sem))
                cps.append(pltpu.async_copy(pz_hbm.at[si_v.at[sl]], sz.at[sl], sem))
                cps.append(pltpu.async_copy(px_hbm.at[di_v.at[sl]], dx.at[sl], sem))
                cps.append(pltpu.async_copy(py_hbm.at[di_v.at[sl]], dy.at[sl], sem))
                cps.append(pltpu.async_copy(pz_hbm.at[di_v.at[sl]], dz.at[sl], sem))
            for cp in cps:
                cp.wait()
            for i in range(_GCH // 16):
                v = pl.ds(i * 16, 16)
                dx[v] = dx[v] - sx[v]
                dy[v] = dy[v] - sy[v]
                dz[v] = dz[v] - sz[v]
            pltpu.sync_copy(dx, ox.at[pl.ds(e0, _GCH)])
            pltpu.sync_copy(dy, oy.at[pl.ds(e0, _GCH)])
            pltpu.sync_copy(dz, oz.at[pl.ds(e0, _GCH)])
            return carry

        lax.fori_loop(0, per_w, body, 0)

    return k(px, py, pz, src1d, dst1d)


def _tc_mlp(relx, rely, relz, w1x, w1y, w1z, b1, g1, be1, W2T, b2, EP):
    BE = 8192
    grid = EP // BE
    rsqrt2 = 0.7071067811865476

    def body(rx_ref, ry_ref, rz_ref, w1x_ref, w1y_ref, w1z_ref, b1_ref,
             g1_ref, be1_ref, w2t_ref, b2_ref, o_ref):
        x = rx_ref[...].reshape(1, BE)
        y = ry_ref[...].reshape(1, BE)
        z = rz_ref[...].reshape(1, BE)
        d = jnp.sqrt(x * x + y * y + z * z) + 1e-6
        ux, uy, uz = x / d, y / d, z / d
        h = (w1x_ref[...] * ux + w1y_ref[...] * uy + w1z_ref[...] * uz
             + b1_ref[...])
        mu = jnp.mean(h, axis=0, keepdims=True)
        hc = h - mu
        var = jnp.mean(hc * hc, axis=0, keepdims=True)
        hn = g1_ref[...] * hc / jnp.sqrt(var + 1e-5) + be1_ref[...]
        hg = 0.5 * hn * (1.0 + lax.erf(hn * rsqrt2))
        code = jnp.dot(w2t_ref[...], hg,
                       preferred_element_type=jnp.float32) + b2_ref[...]
        o_ref[...] = jnp.concatenate(
            [code, jnp.ones((1, BE), jnp.float32)], axis=0)

    full = lambda shape: pl.BlockSpec(shape, lambda i: tuple(0 for _ in shape))
    return pl.pallas_call(
        body,
        grid=(grid,),
        in_specs=[
            pl.BlockSpec((BE,), lambda i: (i,)),
            pl.BlockSpec((BE,), lambda i: (i,)),
            pl.BlockSpec((BE,), lambda i: (i,)),
            full((32, 1)), full((32, 1)), full((32, 1)), full((32, 1)),
            full((32, 1)), full((32, 1)), full((32, 32)), full((32, 1)),
        ],
        out_specs=pl.BlockSpec((_CW, BE), lambda i: (0, i)),
        out_shape=jax.ShapeDtypeStruct((_CW, EP), jnp.float32),
    )(relx, rely, relz, w1x, w1y, w1z, b1, g1, be1, W2T, b2)


def _sc_scatter(codeT4, dst3d, zeros, NP, EP):
    cr = _SCH // 128
    nchunk = EP // _SCH
    per_w = -(-nchunk // _NW)
    mesh = plsc.VectorSubcoreMesh(core_axis_name="c", subcore_axis_name="s")

    @functools.partial(
        pl.kernel,
        out_type=jax.ShapeDtypeStruct((4 * _AW, 1, NP), jnp.float32),
        mesh=mesh,
        scratch_types=[
            pltpu.VMEM((cr, 1, 128), jnp.int32),
            pltpu.VMEM((17, cr, 1, 128), jnp.float32),
            pltpu.VMEM_SHARED((_AW, 1, NP), jnp.float32),
            pltpu.SemaphoreType.DMA,
        ],
    )
    def k(code_hbm, dst_hbm, z_hbm, out_hbm, idx_v, data_v, accum, sem):
        c = lax.axis_index("c")
        s = lax.axis_index("s")
        wid = s * 2 + c

        for p, (f0, nf) in enumerate(((0, 16), (16, 17))):
            @pl.when(s < _AW // 8)
            def _():
                pltpu.sync_copy(z_hbm.at[pl.ds(s * 8, 8)],
                                accum.at[pl.ds(s * 8, 8)])
            plsc.subcore_barrier()

            def body(t, carry):
                ct = wid + t * _NW

                @pl.when(ct < nchunk)
                def _():
                    pltpu.sync_copy(dst_hbm.at[pl.ds(ct * cr, cr)], idx_v)
                    pltpu.sync_copy(
                        code_hbm.at[pl.ds(f0, nf), pl.ds(ct * cr, cr)],
                        data_v.at[pl.ds(0, nf)])
                    cps = []
                    for f in range(nf):
                        for j in range(cr):
                            cps.append(pltpu.async_copy(
                                data_v.at[f, j, 0],
                                accum.at[f, 0].at[idx_v.at[j, 0]],
                                sem, add=True))
                    for cp in cps:
                        cp.wait()
                return carry

            lax.fori_loop(0, per_w, body, 0)
            plsc.subcore_barrier()

            @pl.when(s < _AW // 8)
            def _():
                pltpu.sync_copy(
                    accum.at[pl.ds(s * 8, 8)],
                    out_hbm.at[pl.ds((c * 2 + p) * _AW + s * 8, 8)])
            plsc.subcore_barrier()

    return k(codeT4, dst3d, zeros)


def _tc_combine(x, parts, N, NP):
    BN = 2176
    grid = NP // BN

    def body(x_ref, p_ref, eye_ref, o_ref):
        xb = x_ref[...]
        col = lax.broadcasted_iota(jnp.int32, xb.shape, 1)

        def binf(v, lo, hi):
            vc = jnp.clip(v, lo, hi)
            vn = (vc - lo) / (hi - lo)
            return jnp.floor(vn * 10.0) / 10.0

        xn = jnp.where(col == 0, binf(xb, -4.5, 4.5), xb)
        xn = jnp.where(col == 1, binf(xb, -1.0, 1.0), xn)
        xn = jnp.where(col == 3, binf(xb, 75.0, 204.0), xn)
        p = p_ref[...]
        psA = p[0:_AW] + p[2 * _AW:3 * _AW]
        psB = p[_AW:2 * _AW] + p[3 * _AW:4 * _AW]
        cs = jnp.concatenate([psA[0:16], psB[0:17]], axis=0)
        cnt = cs[32:33]
        peT = cs[0:32] / jnp.maximum(cnt, 1.0)
        pe = lax.dot_general(peT, eye_ref[...],
                             dimension_numbers=(((0,), (0,)), ((), ())),
                             preferred_element_type=jnp.float32)
        o_ref[...] = jnp.concatenate([xn, pe], axis=1)

    eye = jnp.eye(32, dtype=jnp.float32)
    return pl.pallas_call(
        body,
        grid=(grid,),
        in_specs=[
            pl.BlockSpec((BN, 26), lambda i: (i, 0)),
            pl.BlockSpec((4 * _AW, BN), lambda i: (0, i)),
            pl.BlockSpec((32, 32), lambda i: (0, 0)),
        ],
        out_specs=pl.BlockSpec((BN, 58), lambda i: (i, 0)),
        out_shape=jax.ShapeDtypeStruct((N, 58), jnp.float32),
    )(x, parts, eye)


def kernel(x, pos, edge_index, batch, W1, b1, g1, be1, W2, b2):
    N = x.shape[0]
    E = edge_index.shape[1]
    EP = -(-E // (_GCH * _NW)) * (_GCH * _NW)
    NP = N + _TRASH
    npad = EP - E
    src = edge_index[0].astype(jnp.int32)
    dst = edge_index[1].astype(jnp.int32)
    pad_i = jnp.arange(npad, dtype=jnp.int32)
    src_p = jnp.concatenate([src, pad_i % N])
    dst_p = jnp.concatenate([dst, N + (pad_i % _TRASH)])
    dst2d = dst_p.reshape(EP // 128, 128)
    px, py, pz = pos[:, 0], pos[:, 1], pos[:, 2]
    relx, rely, relz = _sc_gather(px, py, pz, src_p, dst_p, EP, N)
    W1T = W1.T
    codeT = _tc_mlp(
        relx, rely, relz,
        W1T[:, 0:1], W1T[:, 1:2], W1T[:, 2:3],
        b1.reshape(32, 1), g1.reshape(32, 1), be1.reshape(32, 1),
        W2.T, b2.reshape(32, 1), EP)
    codeT4 = codeT.reshape(_CW, EP // 128, 1, 128)
    dst3d = dst2d.reshape(EP // 128, 1, 128)
    zeros = jnp.zeros((_AW, 1, NP), jnp.float32)
    parts = _sc_scatter(codeT4, dst3d, zeros, NP, EP)
    return _tc_combine(x, parts.reshape(4 * _AW, NP), N, NP)


# speedup vs baseline: 5.5287x; 1.0288x over previous
import functools

import jax
import jax.numpy as jnp
from jax import lax
from jax.experimental import pallas as pl
from jax.experimental.pallas import tpu as pltpu
from jax.experimental.pallas import tpu_sc as plsc

_CW = 33
_AW = 24
_NW = 32
_GCH = 512
_SCH = 1024
_TRASH = 48


def _sc_gather(pos8, src1d, dst1d, EP, N):
    nchunk = EP // _GCH
    per_w = nchunk // _NW
    nsub = _GCH // 128
    mesh = plsc.VectorSubcoreMesh(core_axis_name="c", subcore_axis_name="s")

    @functools.partial(
        pl.kernel,
        out_type=(jax.ShapeDtypeStruct((EP,), jnp.float32),
                  jax.ShapeDtypeStruct((EP,), jnp.float32),
                  jax.ShapeDtypeStruct((EP,), jnp.float32)),
        mesh=mesh,
        compiler_params=pltpu.CompilerParams(use_tc_tiling_on_sc=False,
                                             needs_layout_passes=False),
        scratch_types=[
            pltpu.VMEM((_GCH,), jnp.int32),
            pltpu.VMEM((_GCH,), jnp.int32),
            pltpu.VMEM((_GCH, 8), jnp.float32),
            pltpu.VMEM((_GCH, 8), jnp.float32),
            pltpu.VMEM((_GCH,), jnp.float32),
            pltpu.VMEM((_GCH,), jnp.float32),
            pltpu.VMEM((_GCH,), jnp.float32),
            pltpu.SemaphoreType.DMA,
        ],
    )
    def k(pos_hbm, src_hbm, dst_hbm, ox, oy, oz,
          si_v, di_v, gs_v, gd_v, rx, ry, rz, sem):
        c = lax.axis_index("c")
        s = lax.axis_index("s")
        wid = s * 2 + c

        def body(t, carry):
            ct = wid * per_w + t
            e0 = ct * _GCH
            pltpu.sync_copy(src_hbm.at[pl.ds(e0, _GCH)], si_v)
            pltpu.sync_copy(dst_hbm.at[pl.ds(e0, _GCH)], di_v)
            cps = []
            for j in range(nsub):
                sl = pl.ds(j * 128, 128)
                cps.append(pltpu.async_copy(
                    pos_hbm.at[si_v.at[sl]], gs_v.at[sl], sem))
                cps.append(pltpu.async_copy(
                    pos_hbm.at[di_v.at[sl]], gd_v.at[sl], sem))
            for cp in cps:
                cp.wait()
            lane = lax.iota(jnp.int32, 16)
            for i in range(_GCH // 16):
                v = pl.ds(i * 16, 16)
                ridx = lane + i * 16
                for col, out in ((0, rx), (1, ry), (2, rz)):
                    cidx = jnp.full((16,), col, jnp.int32)
                    gsv = plsc.load_gather(gs_v, [ridx, cidx])
                    gdv = plsc.load_gather(gd_v, [ridx, cidx])
                    out[v] = gdv - gsv
            pltpu.sync_copy(rx, ox.at[pl.ds(e0, _GCH)])
            pltpu.sync_copy(ry, oy.at[pl.ds(e0, _GCH)])
            pltpu.sync_copy(rz, oz.at[pl.ds(e0, _GCH)])
            return carry

        lax.fori_loop(0, per_w, body, 0)

    return k(pos8, src1d, dst1d)


def _tc_mlp(relx, rely, relz, w1x, w1y, w1z, b1, g1, be1, W2T, b2, EP):
    BE = 8192
    grid = EP // BE
    rsqrt2 = 0.7071067811865476

    def body(rx_ref, ry_ref, rz_ref, w1x_ref, w1y_ref, w1z_ref, b1_ref,
             g1_ref, be1_ref, w2t_ref, b2_ref, o_ref):
        x = rx_ref[...].reshape(1, BE)
        y = ry_ref[...].reshape(1, BE)
        z = rz_ref[...].reshape(1, BE)
        d = jnp.sqrt(x * x + y * y + z * z) + 1e-6
        ux, uy, uz = x / d, y / d, z / d
        h = (w1x_ref[...] * ux + w1y_ref[...] * uy + w1z_ref[...] * uz
             + b1_ref[...])
        mu = jnp.mean(h, axis=0, keepdims=True)
        hc = h - mu
        var = jnp.mean(hc * hc, axis=0, keepdims=True)
        hn = g1_ref[...] * hc / jnp.sqrt(var + 1e-5) + be1_ref[...]
        hg = 0.5 * hn * (1.0 + lax.erf(hn * rsqrt2))
        code = jnp.dot(w2t_ref[...], hg,
                       preferred_element_type=jnp.float32) + b2_ref[...]
        o_ref[...] = jnp.concatenate(
            [code, jnp.ones((1, BE), jnp.float32)], axis=0)

    full = lambda shape: pl.BlockSpec(shape, lambda i: tuple(0 for _ in shape))
    return pl.pallas_call(
        body,
        grid=(grid,),
        in_specs=[
            pl.BlockSpec((BE,), lambda i: (i,)),
            pl.BlockSpec((BE,), lambda i: (i,)),
            pl.BlockSpec((BE,), lambda i: (i,)),
            full((32, 1)), full((32, 1)), full((32, 1)), full((32, 1)),
            full((32, 1)), full((32, 1)), full((32, 32)), full((32, 1)),
        ],
        out_specs=pl.BlockSpec((_CW, BE), lambda i: (0, i)),
        out_shape=jax.ShapeDtypeStruct((_CW, EP), jnp.float32),
    )(relx, rely, relz, w1x, w1y, w1z, b1, g1, be1, W2T, b2)


def _sc_scatter(codeT4, dst3d, zeros, NP, EP):
    cr = _SCH // 128
    nchunk = EP // _SCH
    per_w = -(-nchunk // _NW)
    mesh = plsc.VectorSubcoreMesh(core_axis_name="c", subcore_axis_name="s")

    @functools.partial(
        pl.kernel,
        out_type=jax.ShapeDtypeStruct((4 * _AW, 1, NP), jnp.float32),
        mesh=mesh,
        scratch_types=[
            pltpu.VMEM((cr, 1, 128), jnp.int32),
            pltpu.VMEM((17, cr, 1, 128), jnp.float32),
            pltpu.VMEM_SHARED((_AW, 1, NP), jnp.float32),
            pltpu.SemaphoreType.DMA,
        ],
    )
    def k(code_hbm, dst_hbm, z_hbm, out_hbm, idx_v, data_v, accum, sem):
        c = lax.axis_index("c")
        s = lax.axis_index("s")
        wid = s * 2 + c

        for p, (f0, nf) in enumerate(((0, 16), (16, 17))):
            @pl.when(s < _AW // 8)
            def _():
                pltpu.sync_copy(z_hbm.at[pl.ds(s * 8, 8)],
                                accum.at[pl.ds(s * 8, 8)])
            plsc.subcore_barrier()

            def body(t, carry):
                ct = wid + t * _NW

                @pl.when(ct < nchunk)
                def _():
                    pltpu.sync_copy(dst_hbm.at[pl.ds(ct * cr, cr)], idx_v)
                    pltpu.sync_copy(
                        code_hbm.at[pl.ds(f0, nf), pl.ds(ct * cr, cr)],
                        data_v.at[pl.ds(0, nf)])
                    cps = []
                    for f in range(nf):
                        for j in range(cr):
                            cps.append(pltpu.async_copy(
                                data_v.at[f, j, 0],
                                accum.at[f, 0].at[idx_v.at[j, 0]],
                                sem, add=True))
                    for cp in cps:
                        cp.wait()
                return carry

            lax.fori_loop(0, per_w, body, 0)
            plsc.subcore_barrier()

            @pl.when(s < _AW // 8)
            def _():
                pltpu.sync_copy(
                    accum.at[pl.ds(s * 8, 8)],
                    out_hbm.at[pl.ds((c * 2 + p) * _AW + s * 8, 8)])
            plsc.subcore_barrier()

    return k(codeT4, dst3d, zeros)


def _tc_combine(x, parts, N, NP):
    BN = 2176
    grid = NP // BN

    def body(x_ref, p_ref, eye_ref, o_ref):
        xb = x_ref[...]
        col = lax.broadcasted_iota(jnp.int32, xb.shape, 1)

        def binf(v, lo, hi):
            vc = jnp.clip(v, lo, hi)
            vn = (vc - lo) / (hi - lo)
            return jnp.floor(vn * 10.0) / 10.0

        xn = jnp.where(col == 0, binf(xb, -4.5, 4.5), xb)
        xn = jnp.where(col == 1, binf(xb, -1.0, 1.0), xn)
        xn = jnp.where(col == 3, binf(xb, 75.0, 204.0), xn)
        p = p_ref[...]
        psA = p[0:_AW] + p[2 * _AW:3 * _AW]
        psB = p[_AW:2 * _AW] + p[3 * _AW:4 * _AW]
        cs = jnp.concatenate([psA[0:16], psB[0:17]], axis=0)
        cnt = cs[32:33]
        peT = cs[0:32] / jnp.maximum(cnt, 1.0)
        pe = lax.dot_general(peT, eye_ref[...],
                             dimension_numbers=(((0,), (0,)), ((), ())),
                             preferred_element_type=jnp.float32)
        o_ref[...] = jnp.concatenate([xn, pe], axis=1)

    eye = jnp.eye(32, dtype=jnp.float32)
    return pl.pallas_call(
        body,
        grid=(grid,),
        in_specs=[
            pl.BlockSpec((BN, 26), lambda i: (i, 0)),
            pl.BlockSpec((4 * _AW, BN), lambda i: (0, i)),
            pl.BlockSpec((32, 32), lambda i: (0, 0)),
        ],
        out_specs=pl.BlockSpec((BN, 58), lambda i: (i, 0)),
        out_shape=jax.ShapeDtypeStruct((N, 58), jnp.float32),
    )(x, parts, eye)


def kernel(x, pos, edge_index, batch, W1, b1, g1, be1, W2, b2):
    N = x.shape[0]
    E = edge_index.shape[1]
    EP = -(-E // (_GCH * _NW)) * (_GCH * _NW)
    NP = N + _TRASH
    npad = EP - E
    src = edge_index[0].astype(jnp.int32)
    dst = edge_index[1].astype(jnp.int32)
    pad_i = jnp.arange(npad, dtype=jnp.int32)
    src_p = jnp.concatenate([src, pad_i % N])
    dst_p = jnp.concatenate([dst, N + (pad_i % _TRASH)])
    dst2d = dst_p.reshape(EP // 128, 128)
    pos8 = jnp.pad(pos, ((0, 0), (0, 5)))
    relx, rely, relz = _sc_gather(pos8, src_p, dst_p, EP, N)
    W1T = W1.T
    codeT = _tc_mlp(
        relx, rely, relz,
        W1T[:, 0:1], W1T[:, 1:2], W1T[:, 2:3],
        b1.reshape(32, 1), g1.reshape(32, 1), be1.reshape(32, 1),
        W2.T, b2.reshape(32, 1), EP)
    codeT4 = codeT.reshape(_CW, EP // 128, 1, 128)
    dst3d = dst2d.reshape(EP // 128, 1, 128)
    zeros = jnp.zeros((_AW, 1, NP), jnp.float32)
    parts = _sc_scatter(codeT4, dst3d, zeros, NP, EP)
    return _tc_combine(x, parts.reshape(4 * _AW, NP), N, NP)


# speedup vs baseline: 6.0906x; 1.1016x over previous
import functools

import jax
import jax.numpy as jnp
from jax import lax
from jax.experimental import pallas as pl
from jax.experimental.pallas import tpu as pltpu
from jax.experimental.pallas import tpu_sc as plsc

_CW = 33
_AW = 24
_NW = 32
_GCH = 512
_SCH = 1024
_TRASH = 48


def _sc_gather(px, py, pz, src1d, dst1d, EP, N):
    nchunk = EP // _GCH
    per_w = nchunk // _NW
    nsub = _GCH // 128
    mesh = plsc.VectorSubcoreMesh(core_axis_name="c", subcore_axis_name="s")

    @functools.partial(
        pl.kernel,
        out_type=(jax.ShapeDtypeStruct((EP,), jnp.float32),
                  jax.ShapeDtypeStruct((EP,), jnp.float32),
                  jax.ShapeDtypeStruct((EP,), jnp.float32)),
        mesh=mesh,
        scratch_types=[
            pltpu.VMEM((_GCH,), jnp.int32),
            pltpu.VMEM((_GCH,), jnp.int32),
            pltpu.VMEM((_GCH,), jnp.float32),
            pltpu.VMEM((_GCH,), jnp.float32),
            pltpu.VMEM((_GCH,), jnp.float32),
            pltpu.VMEM((_GCH,), jnp.float32),
            pltpu.VMEM((_GCH,), jnp.float32),
            pltpu.VMEM((_GCH,), jnp.float32),
            pltpu.VMEM_SHARED((N,), jnp.float32),
            pltpu.VMEM_SHARED((N,), jnp.float32),
            pltpu.VMEM_SHARED((N,), jnp.float32),
            pltpu.SemaphoreType.DMA,
        ],
    )
    def k(px_hbm, py_hbm, pz_hbm, src_hbm, dst_hbm, ox, oy, oz,
          si_v, di_v, sx, sy, sz, dx, dy, dz, pxs, pys, pzs, sem):
        c = lax.axis_index("c")
        s = lax.axis_index("s")
        wid = s * 2 + c

        @pl.when(s == 0)
        def _():
            pltpu.sync_copy(px_hbm, pxs)

        @pl.when(s == 1)
        def _():
            pltpu.sync_copy(py_hbm, pys)

        @pl.when(s == 2)
        def _():
            pltpu.sync_copy(pz_hbm, pzs)

        plsc.subcore_barrier()

        def body(t, carry):
            ct = wid * per_w + t
            e0 = ct * _GCH
            pltpu.sync_copy(src_hbm.at[pl.ds(e0, _GCH)], si_v)
            pltpu.sync_copy(dst_hbm.at[pl.ds(e0, _GCH)], di_v)
            cps = []
            for j in range(nsub):
                sl = pl.ds(j * 128, 128)
                cps.append(pltpu.async_copy(pxs.at[si_v.at[sl]], sx.at[sl], sem))
                cps.append(pltpu.async_copy(pys.at[si_v.at[sl]], sy.at[sl], sem))
                cps.append(pltpu.async_copy(pzs.at[si_v.at[sl]], sz.at[sl], sem))
                cps.append(pltpu.async_copy(pxs.at[di_v.at[sl]], dx.at[sl], sem))
                cps.append(pltpu.async_copy(pys.at[di_v.at[sl]], dy.at[sl], sem))
                cps.append(pltpu.async_copy(pzs.at[di_v.at[sl]], dz.at[sl], sem))
            for cp in cps:
                cp.wait()
            for i in range(_GCH // 16):
                v = pl.ds(i * 16, 16)
                dx[v] = dx[v] - sx[v]
                dy[v] = dy[v] - sy[v]
                dz[v] = dz[v] - sz[v]
            pltpu.sync_copy(dx, ox.at[pl.ds(e0, _GCH)])
            pltpu.sync_copy(dy, oy.at[pl.ds(e0, _GCH)])
            pltpu.sync_copy(dz, oz.at[pl.ds(e0, _GCH)])
            return carry

        lax.fori_loop(0, per_w, body, 0)

    return k(px, py, pz, src1d, dst1d)


def _tc_mlp(relx, rely, relz, w1x, w1y, w1z, b1, g1, be1, W2T, b2, EP):
    BE = 8192
    grid = EP // BE
    rsqrt2 = 0.7071067811865476

    def body(rx_ref, ry_ref, rz_ref, w1x_ref, w1y_ref, w1z_ref, b1_ref,
             g1_ref, be1_ref, w2t_ref, b2_ref, o_ref):
        x = rx_ref[...].reshape(1, BE)
        y = ry_ref[...].reshape(1, BE)
        z = rz_ref[...].reshape(1, BE)
        d = jnp.sqrt(x * x + y * y + z * z) + 1e-6
        ux, uy, uz = x / d, y / d, z / d
        h = (w1x_ref[...] * ux + w1y_ref[...] * uy + w1z_ref[...] * uz
             + b1_ref[...])
        mu = jnp.mean(h, axis=0, keepdims=True)
        hc = h - mu
        var = jnp.mean(hc * hc, axis=0, keepdims=True)
        hn = g1_ref[...] * hc / jnp.sqrt(var + 1e-5) + be1_ref[...]
        hg = 0.5 * hn * (1.0 + lax.erf(hn * rsqrt2))
        code = jnp.dot(w2t_ref[...], hg,
                       preferred_element_type=jnp.float32) + b2_ref[...]
        o_ref[...] = jnp.concatenate(
            [code, jnp.ones((1, BE), jnp.float32)], axis=0)

    full = lambda shape: pl.BlockSpec(shape, lambda i: tuple(0 for _ in shape))
    return pl.pallas_call(
        body,
        grid=(grid,),
        in_specs=[
            pl.BlockSpec((BE,), lambda i: (i,)),
            pl.BlockSpec((BE,), lambda i: (i,)),
            pl.BlockSpec((BE,), lambda i: (i,)),
            full((32, 1)), full((32, 1)), full((32, 1)), full((32, 1)),
            full((32, 1)), full((32, 1)), full((32, 32)), full((32, 1)),
        ],
        out_specs=pl.BlockSpec((_CW, BE), lambda i: (0, i)),
        out_shape=jax.ShapeDtypeStruct((_CW, EP), jnp.float32),
    )(relx, rely, relz, w1x, w1y, w1z, b1, g1, be1, W2T, b2)


def _sc_scatter(codeT4, dst3d, zeros, NP, EP):
    cr = _SCH // 128
    nchunk = EP // _SCH
    per_w = -(-nchunk // _NW)
    mesh = plsc.VectorSubcoreMesh(core_axis_name="c", subcore_axis_name="s")

    @functools.partial(
        pl.kernel,
        out_type=jax.ShapeDtypeStruct((4 * _AW, 1, NP), jnp.float32),
        mesh=mesh,
        scratch_types=[
            pltpu.VMEM((cr, 1, 128), jnp.int32),
            pltpu.VMEM((17, cr, 1, 128), jnp.float32),
            pltpu.VMEM_SHARED((_AW, 1, NP), jnp.float32),
            pltpu.SemaphoreType.DMA,
        ],
    )
    def k(code_hbm, dst_hbm, z_hbm, out_hbm, idx_v, data_v, accum, sem):
        c = lax.axis_index("c")
        s = lax.axis_index("s")
        wid = s * 2 + c

        for p, (f0, nf) in enumerate(((0, 16), (16, 17))):
            @pl.when(s < _AW // 8)
            def _():
                pltpu.sync_copy(z_hbm.at[pl.ds(s * 8, 8)],
                                accum.at[pl.ds(s * 8, 8)])
            plsc.subcore_barrier()

            def body(t, carry):
                ct = wid + t * _NW

                @pl.when(ct < nchunk)
                def _():
                    pltpu.sync_copy(dst_hbm.at[pl.ds(ct * cr, cr)], idx_v)
                    pltpu.sync_copy(
                        code_hbm.at[pl.ds(f0, nf), pl.ds(ct * cr, cr)],
                        data_v.at[pl.ds(0, nf)])
                    cps = []
                    for f in range(nf):
                        for j in range(cr):
                            cps.append(pltpu.async_copy(
                                data_v.at[f, j, 0],
                                accum.at[f, 0].at[idx_v.at[j, 0]],
                                sem, add=True))
                    for cp in cps:
                        cp.wait()
                return carry

            lax.fori_loop(0, per_w, body, 0)
            plsc.subcore_barrier()

            @pl.when(s < _AW // 8)
            def _():
                pltpu.sync_copy(
                    accum.at[pl.ds(s * 8, 8)],
                    out_hbm.at[pl.ds((c * 2 + p) * _AW + s * 8, 8)])
            plsc.subcore_barrier()

    return k(codeT4, dst3d, zeros)


def _tc_combine(x, parts, N, NP):
    BN = 2176
    grid = NP // BN

    def body(x_ref, p_ref, eye_ref, o_ref):
        xb = x_ref[...]
        col = lax.broadcasted_iota(jnp.int32, xb.shape, 1)

        def binf(v, lo, hi):
            vc = jnp.clip(v, lo, hi)
            vn = (vc - lo) / (hi - lo)
            return jnp.floor(vn * 10.0) / 10.0

        xn = jnp.where(col == 0, binf(xb, -4.5, 4.5), xb)
        xn = jnp.where(col == 1, binf(xb, -1.0, 1.0), xn)
        xn = jnp.where(col == 3, binf(xb, 75.0, 204.0), xn)
        p = p_ref[...]
        psA = p[0:_AW] + p[2 * _AW:3 * _AW]
        psB = p[_AW:2 * _AW] + p[3 * _AW:4 * _AW]
        cs = jnp.concatenate([psA[0:16], psB[0:17]], axis=0)
        cnt = cs[32:33]
        peT = cs[0:32] / jnp.maximum(cnt, 1.0)
        pe = lax.dot_general(peT, eye_ref[...],
                             dimension_numbers=(((0,), (0,)), ((), ())),
                             preferred_element_type=jnp.float32)
        o_ref[...] = jnp.concatenate([xn, pe], axis=1)

    eye = jnp.eye(32, dtype=jnp.float32)
    return pl.pallas_call(
        body,
        grid=(grid,),
        in_specs=[
            pl.BlockSpec((BN, 26), lambda i: (i, 0)),
            pl.BlockSpec((4 * _AW, BN), lambda i: (0, i)),
            pl.BlockSpec((32, 32), lambda i: (0, 0)),
        ],
        out_specs=pl.BlockSpec((BN, 58), lambda i: (i, 0)),
        out_shape=jax.ShapeDtypeStruct((N, 58), jnp.float32),
    )(x, parts, eye)


def kernel(x, pos, edge_index, batch, W1, b1, g1, be1, W2, b2):
    N = x.shape[0]
    E = edge_index.shape[1]
    EP = -(-E // (_GCH * _NW)) * (_GCH * _NW)
    NP = N + _TRASH
    npad = EP - E
    src = edge_index[0].astype(jnp.int32)
    dst = edge_index[1].astype(jnp.int32)
    pad_i = jnp.arange(npad, dtype=jnp.int32)
    src_p = jnp.concatenate([src, pad_i % N])
    dst_p = jnp.concatenate([dst, N + (pad_i % _TRASH)])
    dst2d = dst_p.reshape(EP // 128, 128)
    px, py, pz = pos[:, 0], pos[:, 1], pos[:, 2]
    relx, rely, relz = _sc_gather(px, py, pz, src_p, dst_p, EP, N)
    W1T = W1.T
    codeT = _tc_mlp(
        relx, rely, relz,
        W1T[:, 0:1], W1T[:, 1:2], W1T[:, 2:3],
        b1.reshape(32, 1), g1.reshape(32, 1), be1.reshape(32, 1),
        W2.T, b2.reshape(32, 1), EP)
    codeT4 = codeT.reshape(_CW, EP // 128, 1, 128)
    dst3d = dst2d.reshape(EP // 128, 1, 128)
    zeros = jnp.zeros((_AW, 1, NP), jnp.float32)
    parts = _sc_scatter(codeT4, dst3d, zeros, NP, EP)
    return _tc_combine(x, parts.reshape(4 * _AW, NP), N, NP)


# speedup vs baseline: 9.1761x; 1.5066x over previous
import functools

import jax
import jax.numpy as jnp
from jax import lax
from jax.experimental import pallas as pl
from jax.experimental.pallas import tpu as pltpu
from jax.experimental.pallas import tpu_sc as plsc

_CW = 33
_AW = 24
_NW = 32
_GCH = 512
_SCH = 1024
_TRASH = 48


def _sc_gather(px, py, pz, src1d, dst1d, EP, N):
    nchunk = EP // _GCH
    per_w = nchunk // _NW
    nsub = _GCH // 128
    mesh = plsc.VectorSubcoreMesh(core_axis_name="c", subcore_axis_name="s")

    @functools.partial(
        pl.kernel,
        out_type=(jax.ShapeDtypeStruct((EP,), jnp.float32),
                  jax.ShapeDtypeStruct((EP,), jnp.float32),
                  jax.ShapeDtypeStruct((EP,), jnp.float32)),
        mesh=mesh,
        scratch_types=[
            pltpu.VMEM((_GCH,), jnp.int32),
            pltpu.VMEM((_GCH,), jnp.int32),
            pltpu.VMEM((_GCH,), jnp.float32),
            pltpu.VMEM((_GCH,), jnp.float32),
            pltpu.VMEM((_GCH,), jnp.float32),
            pltpu.VMEM((_GCH,), jnp.float32),
            pltpu.VMEM((_GCH,), jnp.float32),
            pltpu.VMEM((_GCH,), jnp.float32),
            pltpu.VMEM_SHARED((N,), jnp.float32),
            pltpu.VMEM_SHARED((N,), jnp.float32),
            pltpu.VMEM_SHARED((N,), jnp.float32),
            pltpu.SemaphoreType.DMA,
        ],
    )
    def k(px_hbm, py_hbm, pz_hbm, src_hbm, dst_hbm, ox, oy, oz,
          si_v, di_v, sx, sy, sz, dx, dy, dz, pxs, pys, pzs, sem):
        c = lax.axis_index("c")
        s = lax.axis_index("s")
        wid = s * 2 + c

        @pl.when(s == 0)
        def _():
            pltpu.sync_copy(px_hbm, pxs)

        @pl.when(s == 1)
        def _():
            pltpu.sync_copy(py_hbm, pys)

        @pl.when(s == 2)
        def _():
            pltpu.sync_copy(pz_hbm, pzs)

        plsc.subcore_barrier()

        def body(t, carry):
            ct = wid * per_w + t
            e0 = ct * _GCH
            pltpu.sync_copy(src_hbm.at[pl.ds(e0, _GCH)], si_v)
            pltpu.sync_copy(dst_hbm.at[pl.ds(e0, _GCH)], di_v)
            cps = []
            for j in range(nsub):
                sl = pl.ds(j * 128, 128)
                cps.append(pltpu.async_copy(pxs.at[si_v.at[sl]], sx.at[sl], sem))
                cps.append(pltpu.async_copy(pys.at[si_v.at[sl]], sy.at[sl], sem))
                cps.append(pltpu.async_copy(pzs.at[si_v.at[sl]], sz.at[sl], sem))
                cps.append(pltpu.async_copy(pxs.at[di_v.at[sl]], dx.at[sl], sem))
                cps.append(pltpu.async_copy(pys.at[di_v.at[sl]], dy.at[sl], sem))
                cps.append(pltpu.async_copy(pzs.at[di_v.at[sl]], dz.at[sl], sem))
            for cp in cps:
                cp.wait()
            for i in range(_GCH // 16):
                v = pl.ds(i * 16, 16)
                dx[v] = dx[v] - sx[v]
                dy[v] = dy[v] - sy[v]
                dz[v] = dz[v] - sz[v]
            pltpu.sync_copy(dx, ox.at[pl.ds(e0, _GCH)])
            pltpu.sync_copy(dy, oy.at[pl.ds(e0, _GCH)])
            pltpu.sync_copy(dz, oz.at[pl.ds(e0, _GCH)])
            return carry

        lax.fori_loop(0, per_w, body, 0)

    return k(px, py, pz, src1d, dst1d)


def _tc_mlp(relx, rely, relz, w1x, w1y, w1z, b1, g1, be1, W2T, b2, EP):
    BE = 8192
    grid = EP // BE
    rsqrt2 = 0.7071067811865476

    def body(rx_ref, ry_ref, rz_ref, w1x_ref, w1y_ref, w1z_ref, b1_ref,
             g1_ref, be1_ref, w2t_ref, b2_ref, o_ref):
        x = rx_ref[...].reshape(1, BE)
        y = ry_ref[...].reshape(1, BE)
        z = rz_ref[...].reshape(1, BE)
        d = jnp.sqrt(x * x + y * y + z * z) + 1e-6
        ux, uy, uz = x / d, y / d, z / d
        h = (w1x_ref[...] * ux + w1y_ref[...] * uy + w1z_ref[...] * uz
             + b1_ref[...])
        mu = jnp.mean(h, axis=0, keepdims=True)
        hc = h - mu
        var = jnp.mean(hc * hc, axis=0, keepdims=True)
        hn = g1_ref[...] * hc / jnp.sqrt(var + 1e-5) + be1_ref[...]
        hg = 0.5 * hn * (1.0 + lax.erf(hn * rsqrt2))
        code = jnp.dot(w2t_ref[...], hg,
                       preferred_element_type=jnp.float32) + b2_ref[...]
        o_ref[...] = jnp.concatenate(
            [code, jnp.ones((1, BE), jnp.float32)], axis=0)

    full = lambda shape: pl.BlockSpec(shape, lambda i: tuple(0 for _ in shape))
    return pl.pallas_call(
        body,
        grid=(grid,),
        in_specs=[
            pl.BlockSpec((BE,), lambda i: (i,)),
            pl.BlockSpec((BE,), lambda i: (i,)),
            pl.BlockSpec((BE,), lambda i: (i,)),
            full((32, 1)), full((32, 1)), full((32, 1)), full((32, 1)),
            full((32, 1)), full((32, 1)), full((32, 32)), full((32, 1)),
        ],
        out_specs=pl.BlockSpec((_CW, BE), lambda i: (0, i)),
        out_shape=jax.ShapeDtypeStruct((_CW, EP), jnp.float32),
    )(relx, rely, relz, w1x, w1y, w1z, b1, g1, be1, W2T, b2)


def _sc_scatter(codeT3, dst1d, zeros, NP, EP):
    nchunk = EP // _SCH
    per_w = -(-nchunk // _NW)
    mesh = plsc.VectorSubcoreMesh(core_axis_name="c", subcore_axis_name="s")

    @functools.partial(
        pl.kernel,
        out_type=jax.ShapeDtypeStruct((4 * _AW, 1, NP), jnp.float32),
        mesh=mesh,
        scratch_types=[
            pltpu.VMEM((_SCH,), jnp.int32),
            pltpu.VMEM((17, 1, _SCH), jnp.float32),
            pltpu.VMEM_SHARED((_AW, 1, NP), jnp.float32),
            pltpu.SemaphoreType.DMA,
        ],
    )
    def k(code_hbm, dst_hbm, z_hbm, out_hbm, idx_v, data_v, accum, sem):
        c = lax.axis_index("c")
        s = lax.axis_index("s")
        wid = s * 2 + c

        for p, (f0, nf) in enumerate(((0, 16), (16, 17))):
            @pl.when(s < _AW // 8)
            def _():
                pltpu.sync_copy(z_hbm.at[pl.ds(s * 8, 8)],
                                accum.at[pl.ds(s * 8, 8)])
            plsc.subcore_barrier()

            def body(t, carry):
                ct = wid + t * _NW

                @pl.when(ct < nchunk)
                def _():
                    pltpu.sync_copy(dst_hbm.at[pl.ds(ct * _SCH, _SCH)], idx_v)
                    pltpu.sync_copy(
                        code_hbm.at[pl.ds(f0, nf), :, pl.ds(ct * _SCH, _SCH)],
                        data_v.at[pl.ds(0, nf)])
                    cps = []
                    for f in range(nf):
                        cps.append(pltpu.async_copy(
                            data_v.at[f, 0],
                            accum.at[f, 0].at[idx_v],
                            sem, add=True))
                    for cp in cps:
                        cp.wait()
                return carry

            lax.fori_loop(0, per_w, body, 0)
            plsc.subcore_barrier()

            @pl.when(s < _AW // 8)
            def _():
                pltpu.sync_copy(
                    accum.at[pl.ds(s * 8, 8)],
                    out_hbm.at[pl.ds((c * 2 + p) * _AW + s * 8, 8)])
            plsc.subcore_barrier()

    return k(codeT3, dst1d, zeros)


def _tc_combine(x, parts, N, NP):
    BN = 2176
    grid = NP // BN

    def body(x_ref, p_ref, eye_ref, o_ref):
        xb = x_ref[...]
        col = lax.broadcasted_iota(jnp.int32, xb.shape, 1)

        def binf(v, lo, hi):
            vc = jnp.clip(v, lo, hi)
            vn = (vc - lo) / (hi - lo)
            return jnp.floor(vn * 10.0) / 10.0

        xn = jnp.where(col == 0, binf(xb, -4.5, 4.5), xb)
        xn = jnp.where(col == 1, binf(xb, -1.0, 1.0), xn)
        xn = jnp.where(col == 3, binf(xb, 75.0, 204.0), xn)
        p = p_ref[...]
        psA = p[0:_AW] + p[2 * _AW:3 * _AW]
        psB = p[_AW:2 * _AW] + p[3 * _AW:4 * _AW]
        cs = jnp.concatenate([psA[0:16], psB[0:17]], axis=0)
        cnt = cs[32:33]
        peT = cs[0:32] / jnp.maximum(cnt, 1.0)
        pe = lax.dot_general(peT, eye_ref[...],
                             dimension_numbers=(((0,), (0,)), ((), ())),
                             preferred_element_type=jnp.float32)
        o_ref[...] = jnp.concatenate([xn, pe], axis=1)

    eye = jnp.eye(32, dtype=jnp.float32)
    return pl.pallas_call(
        body,
        grid=(grid,),
        in_specs=[
            pl.BlockSpec((BN, 26), lambda i: (i, 0)),
            pl.BlockSpec((4 * _AW, BN), lambda i: (0, i)),
            pl.BlockSpec((32, 32), lambda i: (0, 0)),
        ],
        out_specs=pl.BlockSpec((BN, 58), lambda i: (i, 0)),
        out_shape=jax.ShapeDtypeStruct((N, 58), jnp.float32),
    )(x, parts, eye)


def kernel(x, pos, edge_index, batch, W1, b1, g1, be1, W2, b2):
    N = x.shape[0]
    E = edge_index.shape[1]
    EP = -(-E // (_GCH * _NW)) * (_GCH * _NW)
    NP = N + _TRASH
    npad = EP - E
    src = edge_index[0].astype(jnp.int32)
    dst = edge_index[1].astype(jnp.int32)
    pad_i = jnp.arange(npad, dtype=jnp.int32)
    src_p = jnp.concatenate([src, pad_i % N])
    dst_p = jnp.concatenate([dst, N + (pad_i % _TRASH)])
    dst2d = dst_p.reshape(EP // 128, 128)
    px, py, pz = pos[:, 0], pos[:, 1], pos[:, 2]
    relx, rely, relz = _sc_gather(px, py, pz, src_p, dst_p, EP, N)
    W1T = W1.T
    codeT = _tc_mlp(
        relx, rely, relz,
        W1T[:, 0:1], W1T[:, 1:2], W1T[:, 2:3],
        b1.reshape(32, 1), g1.reshape(32, 1), be1.reshape(32, 1),
        W2.T, b2.reshape(32, 1), EP)
    codeT3 = codeT.reshape(_CW, 1, EP)
    zeros = jnp.zeros((_AW, 1, NP), jnp.float32)
    parts = _sc_scatter(codeT3, dst_p, zeros, NP, EP)
    return _tc_combine(x, parts.reshape(4 * _AW, NP), N, NP)


# speedup vs baseline: 9.3445x; 1.0184x over previous
import functools

import jax
import jax.numpy as jnp
from jax import lax
from jax.experimental import pallas as pl
from jax.experimental.pallas import tpu as pltpu
from jax.experimental.pallas import tpu_sc as plsc

_CW = 33
_AW = 24
_NW = 32
_GCH = 1024
_SCH = 1024
_TRASH = 48


def _sc_gather(px, py, pz, src1d, dst1d, EP, N):
    nchunk = EP // _GCH
    per_w = nchunk // _NW
    nsub = _GCH // 128
    mesh = plsc.VectorSubcoreMesh(core_axis_name="c", subcore_axis_name="s")

    @functools.partial(
        pl.kernel,
        out_type=(jax.ShapeDtypeStruct((EP,), jnp.float32),
                  jax.ShapeDtypeStruct((EP,), jnp.float32),
                  jax.ShapeDtypeStruct((EP,), jnp.float32)),
        mesh=mesh,
        scratch_types=[
            pltpu.VMEM((_GCH,), jnp.int32),
            pltpu.VMEM((_GCH,), jnp.int32),
            pltpu.VMEM((_GCH,), jnp.float32),
            pltpu.VMEM((_GCH,), jnp.float32),
            pltpu.VMEM((_GCH,), jnp.float32),
            pltpu.VMEM((_GCH,), jnp.float32),
            pltpu.VMEM((_GCH,), jnp.float32),
            pltpu.VMEM((_GCH,), jnp.float32),
            pltpu.VMEM_SHARED((N,), jnp.float32),
            pltpu.VMEM_SHARED((N,), jnp.float32),
            pltpu.VMEM_SHARED((N,), jnp.float32),
            pltpu.SemaphoreType.DMA,
        ],
    )
    def k(px_hbm, py_hbm, pz_hbm, src_hbm, dst_hbm, ox, oy, oz,
          si_v, di_v, sx, sy, sz, dx, dy, dz, pxs, pys, pzs, sem):
        c = lax.axis_index("c")
        s = lax.axis_index("s")
        wid = s * 2 + c

        @pl.when(s == 0)
        def _():
            pltpu.sync_copy(px_hbm, pxs)

        @pl.when(s == 1)
        def _():
            pltpu.sync_copy(py_hbm, pys)

        @pl.when(s == 2)
        def _():
            pltpu.sync_copy(pz_hbm, pzs)

        plsc.subcore_barrier()

        def body(t, carry):
            ct = wid * per_w + t
            e0 = ct * _GCH
            pltpu.sync_copy(src_hbm.at[pl.ds(e0, _GCH)], si_v)
            pltpu.sync_copy(dst_hbm.at[pl.ds(e0, _GCH)], di_v)
            cps = [
                pltpu.async_copy(pxs.at[si_v], sx, sem),
                pltpu.async_copy(pys.at[si_v], sy, sem),
                pltpu.async_copy(pzs.at[si_v], sz, sem),
                pltpu.async_copy(pxs.at[di_v], dx, sem),
                pltpu.async_copy(pys.at[di_v], dy, sem),
                pltpu.async_copy(pzs.at[di_v], dz, sem),
            ]
            for cp in cps:
                cp.wait()
            for i in range(_GCH // 16):
                v = pl.ds(i * 16, 16)
                dx[v] = dx[v] - sx[v]
                dy[v] = dy[v] - sy[v]
                dz[v] = dz[v] - sz[v]
            pltpu.sync_copy(dx, ox.at[pl.ds(e0, _GCH)])
            pltpu.sync_copy(dy, oy.at[pl.ds(e0, _GCH)])
            pltpu.sync_copy(dz, oz.at[pl.ds(e0, _GCH)])
            return carry

        lax.fori_loop(0, per_w, body, 0)

    return k(px, py, pz, src1d, dst1d)


def _tc_mlp(relx, rely, relz, w1x, w1y, w1z, b1, g1, be1, W2T, b2, EP):
    BE = 8192
    grid = EP // BE
    rsqrt2 = 0.7071067811865476

    def body(rx_ref, ry_ref, rz_ref, w1x_ref, w1y_ref, w1z_ref, b1_ref,
             g1_ref, be1_ref, w2t_ref, b2_ref, o_ref):
        x = rx_ref[...].reshape(1, BE)
        y = ry_ref[...].reshape(1, BE)
        z = rz_ref[...].reshape(1, BE)
        d = jnp.sqrt(x * x + y * y + z * z) + 1e-6
        ux, uy, uz = x / d, y / d, z / d
        h = (w1x_ref[...] * ux + w1y_ref[...] * uy + w1z_ref[...] * uz
             + b1_ref[...])
        mu = jnp.mean(h, axis=0, keepdims=True)
        hc = h - mu
        var = jnp.mean(hc * hc, axis=0, keepdims=True)
        hn = g1_ref[...] * hc / jnp.sqrt(var + 1e-5) + be1_ref[...]
        hg = 0.5 * hn * (1.0 + lax.erf(hn * rsqrt2))
        code = jnp.dot(w2t_ref[...], hg,
                       preferred_element_type=jnp.float32) + b2_ref[...]
        o_ref[...] = jnp.concatenate(
            [code, jnp.ones((1, BE), jnp.float32)], axis=0)

    full = lambda shape: pl.BlockSpec(shape, lambda i: tuple(0 for _ in shape))
    return pl.pallas_call(
        body,
        grid=(grid,),
        in_specs=[
            pl.BlockSpec((BE,), lambda i: (i,)),
            pl.BlockSpec((BE,), lambda i: (i,)),
            pl.BlockSpec((BE,), lambda i: (i,)),
            full((32, 1)), full((32, 1)), full((32, 1)), full((32, 1)),
            full((32, 1)), full((32, 1)), full((32, 32)), full((32, 1)),
        ],
        out_specs=pl.BlockSpec((_CW, BE), lambda i: (0, i)),
        out_shape=jax.ShapeDtypeStruct((_CW, EP), jnp.float32),
    )(relx, rely, relz, w1x, w1y, w1z, b1, g1, be1, W2T, b2)


def _sc_scatter(codeT3, dst1d, zeros, NP, EP):
    nchunk = EP // _SCH
    per_w = -(-nchunk // _NW)
    mesh = plsc.VectorSubcoreMesh(core_axis_name="c", subcore_axis_name="s")

    @functools.partial(
        pl.kernel,
        out_type=jax.ShapeDtypeStruct((4 * _AW, 1, NP), jnp.float32),
        mesh=mesh,
        scratch_types=[
            pltpu.VMEM((_SCH,), jnp.int32),
            pltpu.VMEM((17, 1, _SCH), jnp.float32),
            pltpu.VMEM_SHARED((_AW, 1, NP), jnp.float32),
            pltpu.SemaphoreType.DMA,
        ],
    )
    def k(code_hbm, dst_hbm, z_hbm, out_hbm, idx_v, data_v, accum, sem):
        c = lax.axis_index("c")
        s = lax.axis_index("s")
        wid = s * 2 + c

        for p, (f0, nf) in enumerate(((0, 16), (16, 17))):
            @pl.when(s < _AW // 8)
            def _():
                pltpu.sync_copy(z_hbm.at[pl.ds(s * 8, 8)],
                                accum.at[pl.ds(s * 8, 8)])
            plsc.subcore_barrier()

            def body(t, carry):
                ct = wid + t * _NW

                @pl.when(ct < nchunk)
                def _():
                    pltpu.sync_copy(dst_hbm.at[pl.ds(ct * _SCH, _SCH)], idx_v)
                    pltpu.sync_copy(
                        code_hbm.at[pl.ds(f0, nf), :, pl.ds(ct * _SCH, _SCH)],
                        data_v.at[pl.ds(0, nf)])
                    cps = []
                    for f in range(nf):
                        cps.append(pltpu.async_copy(
                            data_v.at[f, 0],
                            accum.at[f, 0].at[idx_v],
                            sem, add=True))
                    for cp in cps:
                        cp.wait()
                return carry

            lax.fori_loop(0, per_w, body, 0)
            plsc.subcore_barrier()

            @pl.when(s < _AW // 8)
            def _():
                pltpu.sync_copy(
                    accum.at[pl.ds(s * 8, 8)],
                    out_hbm.at[pl.ds((c * 2 + p) * _AW + s * 8, 8)])
            plsc.subcore_barrier()

    return k(codeT3, dst1d, zeros)


def _tc_combine(x, parts, N, NP):
    BN = 2176
    grid = NP // BN

    def body(x_ref, p_ref, eye_ref, o_ref):
        xb = x_ref[...]
        col = lax.broadcasted_iota(jnp.int32, xb.shape, 1)

        def binf(v, lo, hi):
            vc = jnp.clip(v, lo, hi)
            vn = (vc - lo) / (hi - lo)
            return jnp.floor(vn * 10.0) / 10.0

        xn = jnp.where(col == 0, binf(xb, -4.5, 4.5), xb)
        xn = jnp.where(col == 1, binf(xb, -1.0, 1.0), xn)
        xn = jnp.where(col == 3, binf(xb, 75.0, 204.0), xn)
        p = p_ref[...]
        psA = p[0:_AW] + p[2 * _AW:3 * _AW]
        psB = p[_AW:2 * _AW] + p[3 * _AW:4 * _AW]
        cs = jnp.concatenate([psA[0:16], psB[0:17]], axis=0)
        cnt = cs[32:33]
        peT = cs[0:32] / jnp.maximum(cnt, 1.0)
        pe = lax.dot_general(peT, eye_ref[...],
                             dimension_numbers=(((0,), (0,)), ((), ())),
                             preferred_element_type=jnp.float32)
        o_ref[...] = jnp.concatenate([xn, pe], axis=1)

    eye = jnp.eye(32, dtype=jnp.float32)
    return pl.pallas_call(
        body,
        grid=(grid,),
        in_specs=[
            pl.BlockSpec((BN, 26), lambda i: (i, 0)),
            pl.BlockSpec((4 * _AW, BN), lambda i: (0, i)),
            pl.BlockSpec((32, 32), lambda i: (0, 0)),
        ],
        out_specs=pl.BlockSpec((BN, 58), lambda i: (i, 0)),
        out_shape=jax.ShapeDtypeStruct((N, 58), jnp.float32),
    )(x, parts, eye)


def kernel(x, pos, edge_index, batch, W1, b1, g1, be1, W2, b2):
    N = x.shape[0]
    E = edge_index.shape[1]
    EP = -(-E // (_GCH * _NW)) * (_GCH * _NW)
    NP = N + _TRASH
    npad = EP - E
    src = edge_index[0].astype(jnp.int32)
    dst = edge_index[1].astype(jnp.int32)
    pad_i = jnp.arange(npad, dtype=jnp.int32)
    src_p = jnp.concatenate([src, pad_i % N])
    dst_p = jnp.concatenate([dst, N + (pad_i % _TRASH)])
    dst2d = dst_p.reshape(EP // 128, 128)
    px, py, pz = pos[:, 0], pos[:, 1], pos[:, 2]
    relx, rely, relz = _sc_gather(px, py, pz, src_p, dst_p, EP, N)
    W1T = W1.T
    codeT = _tc_mlp(
        relx, rely, relz,
        W1T[:, 0:1], W1T[:, 1:2], W1T[:, 2:3],
        b1.reshape(32, 1), g1.reshape(32, 1), be1.reshape(32, 1),
        W2.T, b2.reshape(32, 1), EP)
    codeT3 = codeT.reshape(_CW, 1, EP)
    zeros = jnp.zeros((_AW, 1, NP), jnp.float32)
    parts = _sc_scatter(codeT3, dst_p, zeros, NP, EP)
    return _tc_combine(x, parts.reshape(4 * _AW, NP), N, NP)


# speedup vs baseline: 9.5586x; 1.0229x over previous
import functools

import jax
import jax.numpy as jnp
from jax import lax
from jax.experimental import pallas as pl
from jax.experimental.pallas import tpu as pltpu
from jax.experimental.pallas import tpu_sc as plsc

_CW = 33
_AW = 24
_NW = 32
_GCH = 1024
_SCH = 2048
_TRASH = 48


def _sc_gather(px, py, pz, src1d, dst1d, EP, N):
    nchunk = EP // _GCH
    per_w = nchunk // _NW
    nsub = _GCH // 128
    mesh = plsc.VectorSubcoreMesh(core_axis_name="c", subcore_axis_name="s")

    @functools.partial(
        pl.kernel,
        out_type=(jax.ShapeDtypeStruct((EP,), jnp.float32),
                  jax.ShapeDtypeStruct((EP,), jnp.float32),
                  jax.ShapeDtypeStruct((EP,), jnp.float32)),
        mesh=mesh,
        scratch_types=[
            pltpu.VMEM((_GCH,), jnp.int32),
            pltpu.VMEM((_GCH,), jnp.int32),
            pltpu.VMEM((_GCH,), jnp.float32),
            pltpu.VMEM((_GCH,), jnp.float32),
            pltpu.VMEM((_GCH,), jnp.float32),
            pltpu.VMEM((_GCH,), jnp.float32),
            pltpu.VMEM((_GCH,), jnp.float32),
            pltpu.VMEM((_GCH,), jnp.float32),
            pltpu.VMEM_SHARED((N,), jnp.float32),
            pltpu.VMEM_SHARED((N,), jnp.float32),
            pltpu.VMEM_SHARED((N,), jnp.float32),
            pltpu.SemaphoreType.DMA,
        ],
    )
    def k(px_hbm, py_hbm, pz_hbm, src_hbm, dst_hbm, ox, oy, oz,
          si_v, di_v, sx, sy, sz, dx, dy, dz, pxs, pys, pzs, sem):
        c = lax.axis_index("c")
        s = lax.axis_index("s")
        wid = s * 2 + c

        @pl.when(s == 0)
        def _():
            pltpu.sync_copy(px_hbm, pxs)

        @pl.when(s == 1)
        def _():
            pltpu.sync_copy(py_hbm, pys)

        @pl.when(s == 2)
        def _():
            pltpu.sync_copy(pz_hbm, pzs)

        plsc.subcore_barrier()

        def body(t, carry):
            ct = wid * per_w + t
            e0 = ct * _GCH
            pltpu.sync_copy(src_hbm.at[pl.ds(e0, _GCH)], si_v)
            pltpu.sync_copy(dst_hbm.at[pl.ds(e0, _GCH)], di_v)
            cps = [
                pltpu.async_copy(pxs.at[si_v], sx, sem),
                pltpu.async_copy(pys.at[si_v], sy, sem),
                pltpu.async_copy(pzs.at[si_v], sz, sem),
                pltpu.async_copy(pxs.at[di_v], dx, sem),
                pltpu.async_copy(pys.at[di_v], dy, sem),
                pltpu.async_copy(pzs.at[di_v], dz, sem),
            ]
            for cp in cps:
                cp.wait()
            for i in range(_GCH // 16):
                v = pl.ds(i * 16, 16)
                dx[v] = dx[v] - sx[v]
                dy[v] = dy[v] - sy[v]
                dz[v] = dz[v] - sz[v]
            pltpu.sync_copy(dx, ox.at[pl.ds(e0, _GCH)])
            pltpu.sync_copy(dy, oy.at[pl.ds(e0, _GCH)])
            pltpu.sync_copy(dz, oz.at[pl.ds(e0, _GCH)])
            return carry

        lax.fori_loop(0, per_w, body, 0)

    return k(px, py, pz, src1d, dst1d)


def _tc_mlp(relx, rely, relz, w1x, w1y, w1z, b1, g1, be1, W2T, b2, EP):
    BE = 8192
    grid = EP // BE
    rsqrt2 = 0.7071067811865476

    def body(rx_ref, ry_ref, rz_ref, w1x_ref, w1y_ref, w1z_ref, b1_ref,
             g1_ref, be1_ref, w2t_ref, b2_ref, o_ref):
        x = rx_ref[...].reshape(1, BE)
        y = ry_ref[...].reshape(1, BE)
        z = rz_ref[...].reshape(1, BE)
        d = jnp.sqrt(x * x + y * y + z * z) + 1e-6
        ux, uy, uz = x / d, y / d, z / d
        h = (w1x_ref[...] * ux + w1y_ref[...] * uy + w1z_ref[...] * uz
             + b1_ref[...])
        mu = jnp.mean(h, axis=0, keepdims=True)
        hc = h - mu
        var = jnp.mean(hc * hc, axis=0, keepdims=True)
        hn = g1_ref[...] * hc / jnp.sqrt(var + 1e-5) + be1_ref[...]
        hg = 0.5 * hn * (1.0 + lax.erf(hn * rsqrt2))
        code = jnp.dot(w2t_ref[...], hg,
                       preferred_element_type=jnp.float32) + b2_ref[...]
        o_ref[...] = jnp.concatenate(
            [code, jnp.ones((1, BE), jnp.float32)], axis=0)

    full = lambda shape: pl.BlockSpec(shape, lambda i: tuple(0 for _ in shape))
    return pl.pallas_call(
        body,
        grid=(grid,),
        in_specs=[
            pl.BlockSpec((BE,), lambda i: (i,)),
            pl.BlockSpec((BE,), lambda i: (i,)),
            pl.BlockSpec((BE,), lambda i: (i,)),
            full((32, 1)), full((32, 1)), full((32, 1)), full((32, 1)),
            full((32, 1)), full((32, 1)), full((32, 32)), full((32, 1)),
        ],
        out_specs=pl.BlockSpec((_CW, BE), lambda i: (0, i)),
        out_shape=jax.ShapeDtypeStruct((_CW, EP), jnp.float32),
    )(relx, rely, relz, w1x, w1y, w1z, b1, g1, be1, W2T, b2)


def _sc_scatter(codeT3, dst1d, zeros, NP, EP):
    nchunk = EP // _SCH
    per_w = -(-nchunk // _NW)
    mesh = plsc.VectorSubcoreMesh(core_axis_name="c", subcore_axis_name="s")

    @functools.partial(
        pl.kernel,
        out_type=jax.ShapeDtypeStruct((4 * _AW, 1, NP), jnp.float32),
        mesh=mesh,
        scratch_types=[
            pltpu.VMEM((_SCH,), jnp.int32),
            pltpu.VMEM((17, 1, _SCH), jnp.float32),
            pltpu.VMEM_SHARED((_AW, 1, NP), jnp.float32),
            pltpu.SemaphoreType.DMA,
        ],
    )
    def k(code_hbm, dst_hbm, z_hbm, out_hbm, idx_v, data_v, accum, sem):
        c = lax.axis_index("c")
        s = lax.axis_index("s")
        wid = s * 2 + c

        for p, (f0, nf) in enumerate(((0, 16), (16, 17))):
            @pl.when(s < _AW // 8)
            def _():
                pltpu.sync_copy(z_hbm.at[pl.ds(s * 8, 8)],
                                accum.at[pl.ds(s * 8, 8)])
            plsc.subcore_barrier()

            def body(t, carry):
                ct = wid + t * _NW

                @pl.when(ct < nchunk)
                def _():
                    pltpu.sync_copy(dst_hbm.at[pl.ds(ct * _SCH, _SCH)], idx_v)
                    pltpu.sync_copy(
                        code_hbm.at[pl.ds(f0, nf), :, pl.ds(ct * _SCH, _SCH)],
                        data_v.at[pl.ds(0, nf)])
                    cps = []
                    for f in range(nf):
                        cps.append(pltpu.async_copy(
                            data_v.at[f, 0],
                            accum.at[f, 0].at[idx_v],
                            sem, add=True))
                    for cp in cps:
                        cp.wait()
                return carry

            lax.fori_loop(0, per_w, body, 0)
            plsc.subcore_barrier()

            @pl.when(s < _AW // 8)
            def _():
                pltpu.sync_copy(
                    accum.at[pl.ds(s * 8, 8)],
                    out_hbm.at[pl.ds((c * 2 + p) * _AW + s * 8, 8)])
            plsc.subcore_barrier()

    return k(codeT3, dst1d, zeros)


def _tc_combine(x, parts, N, NP):
    BN = 2176
    grid = NP // BN

    def body(x_ref, p_ref, eye_ref, o_ref):
        xb = x_ref[...]
        col = lax.broadcasted_iota(jnp.int32, xb.shape, 1)

        def binf(v, lo, hi):
            vc = jnp.clip(v, lo, hi)
            vn = (vc - lo) / (hi - lo)
            return jnp.floor(vn * 10.0) / 10.0

        xn = jnp.where(col == 0, binf(xb, -4.5, 4.5), xb)
        xn = jnp.where(col == 1, binf(xb, -1.0, 1.0), xn)
        xn = jnp.where(col == 3, binf(xb, 75.0, 204.0), xn)
        p = p_ref[...]
        psA = p[0:_AW] + p[2 * _AW:3 * _AW]
        psB = p[_AW:2 * _AW] + p[3 * _AW:4 * _AW]
        cs = jnp.concatenate([psA[0:16], psB[0:17]], axis=0)
        cnt = cs[32:33]
        peT = cs[0:32] / jnp.maximum(cnt, 1.0)
        pe = lax.dot_general(peT, eye_ref[...],
                             dimension_numbers=(((0,), (0,)), ((), ())),
                             preferred_element_type=jnp.float32)
        o_ref[...] = jnp.concatenate([xn, pe], axis=1)

    eye = jnp.eye(32, dtype=jnp.float32)
    return pl.pallas_call(
        body,
        grid=(grid,),
        in_specs=[
            pl.BlockSpec((BN, 26), lambda i: (i, 0)),
            pl.BlockSpec((4 * _AW, BN), lambda i: (0, i)),
            pl.BlockSpec((32, 32), lambda i: (0, 0)),
        ],
        out_specs=pl.BlockSpec((BN, 58), lambda i: (i, 0)),
        out_shape=jax.ShapeDtypeStruct((N, 58), jnp.float32),
    )(x, parts, eye)


def kernel(x, pos, edge_index, batch, W1, b1, g1, be1, W2, b2):
    N = x.shape[0]
    E = edge_index.shape[1]
    EP = -(-E // (_GCH * _NW)) * (_GCH * _NW)
    NP = N + _TRASH
    npad = EP - E
    src = edge_index[0].astype(jnp.int32)
    dst = edge_index[1].astype(jnp.int32)
    pad_i = jnp.arange(npad, dtype=jnp.int32)
    src_p = jnp.concatenate([src, pad_i % N])
    dst_p = jnp.concatenate([dst, N + (pad_i % _TRASH)])
    dst2d = dst_p.reshape(EP // 128, 128)
    px, py, pz = pos[:, 0], pos[:, 1], pos[:, 2]
    relx, rely, relz = _sc_gather(px, py, pz, src_p, dst_p, EP, N)
    W1T = W1.T
    codeT = _tc_mlp(
        relx, rely, relz,
        W1T[:, 0:1], W1T[:, 1:2], W1T[:, 2:3],
        b1.reshape(32, 1), g1.reshape(32, 1), be1.reshape(32, 1),
        W2.T, b2.reshape(32, 1), EP)
    codeT3 = codeT.reshape(_CW, 1, EP)
    zeros = jnp.zeros((_AW, 1, NP), jnp.float32)
    parts = _sc_scatter(codeT3, dst_p, zeros, NP, EP)
    return _tc_combine(x, parts.reshape(4 * _AW, NP), N, NP)
